# baseline (speedup 1.0000x reference)
import functools

import numpy as np

import jax
import jax.numpy as jnp
from jax.experimental import pallas as pl
from jax.experimental.pallas import tpu as pltpu
from jax.sharding import Mesh, PartitionSpec as P

try:
    from jax import shard_map as _shard_map
except ImportError:
    from jax.experimental.shard_map import shard_map as _shard_map


def _cdiv(a, b):
    return -(-a // b)


def _ffn_kernel(x_ref, w1_ref, b1_ref, w2_ref, b2_ref, o_ref):
    h = jnp.dot(x_ref[...], w1_ref[...], preferred_element_type=jnp.float32)
    h = jnp.maximum(h + b1_ref[...], 0.0)
    y = jnp.dot(h, w2_ref[...], preferred_element_type=jnp.float32)
    o_ref[...] = y + b2_ref[...]


def _ffn_local(x, w1, b1, w2, b2, *, block_m=512):
    batch, seq, d_model = x.shape
    hidden = w1.shape[1]
    M = batch * seq

    x2d = x.reshape(M, d_model)
    bm = min(block_m, M)
    n_m = _cdiv(M, bm)

    out2d = pl.pallas_call(
        _ffn_kernel,
        out_shape=jax.ShapeDtypeStruct((M, d_model), jnp.float32),
        grid=(n_m,),
        in_specs=[
            pl.BlockSpec((bm, d_model), lambda i: (i, 0)),
            pl.BlockSpec((d_model, hidden), lambda i: (0, 0)),
            pl.BlockSpec((1, hidden), lambda i: (0, 0)),
            pl.BlockSpec((hidden, d_model), lambda i: (0, 0)),
            pl.BlockSpec((1, d_model), lambda i: (0, 0)),
        ],
        out_specs=pl.BlockSpec((bm, d_model), lambda i: (i, 0)),
        compiler_params=pltpu.CompilerParams(
            dimension_semantics=("parallel",),
            vmem_limit_bytes=int(0.95 * 64 * 1024 * 1024),
        ),
    )(x2d, w1, b1, w2, b2)

    return out2d.reshape(batch, seq, d_model)


_DEVICES = jax.devices()
_N_SHARDS = 2 if len(_DEVICES) >= 2 else 1
_MESH = Mesh(np.array(_DEVICES[:_N_SHARDS]), ("d",))


def kernel(x, w1, b1, w2, b2):
    if _N_SHARDS == 1 or x.shape[0] % _N_SHARDS != 0:
        return _ffn_local(x, w1, b1, w2, b2)
    sharded = _shard_map(
        _ffn_local,
        mesh=_MESH,
        in_specs=(P("d"), P(), P(), P(), P()),
        out_specs=P("d"),
        check_vma=False,
    )
    return sharded(x, w1, b1, w2, b2)


# speedup vs baseline: 5.9450x; 5.9450x over previous
import functools

import jax
import jax.numpy as jnp
from jax.experimental import pallas as pl
from jax.experimental.pallas import tpu as pltpu


def _cdiv(a, b):
    return -(-a // b)


def _ffn_kernel(x_ref, w1_ref, b1_ref, w2_hbm, b2_ref, o_ref, w2_v, sem):
    hidden = w2_v.shape[0]
    half = hidden // 2
    first = pl.program_id(0) == 0

    @pl.when(first)
    def _():
        pltpu.make_async_copy(w2_hbm.at[0:half], w2_v.at[0:half], sem.at[0]).start()
        pltpu.make_async_copy(w2_hbm.at[half:hidden], w2_v.at[half:hidden],
                              sem.at[1]).start()

    h = jnp.dot(x_ref[...], w1_ref[...], preferred_element_type=jnp.float32)
    h = jnp.maximum(h + b1_ref[...], 0.0)

    @pl.when(first)
    def _():
        pltpu.make_async_copy(w2_v.at[0:half], w2_v.at[0:half], sem.at[0]).wait()

    y = jnp.dot(h[:, 0:half], w2_v[0:half, :], preferred_element_type=jnp.float32)

    @pl.when(first)
    def _():
        pltpu.make_async_copy(w2_v.at[half:hidden], w2_v.at[half:hidden],
                              sem.at[1]).wait()

    y = y + jnp.dot(h[:, half:], w2_v[half:, :], preferred_element_type=jnp.float32)
    o_ref[...] = y + b2_ref[...]


@functools.partial(jax.jit, static_argnames=("block_m",))
def _ffn(x, w1, b1, w2, b2, *, block_m=512):
    batch, seq, d_model = x.shape
    hidden = w1.shape[1]
    M = batch * seq

    x2d = x.reshape(M, d_model)
    bm = min(block_m, M)
    n_m = _cdiv(M, bm)

    out2d = pl.pallas_call(
        _ffn_kernel,
        out_shape=jax.ShapeDtypeStruct((M, d_model), jnp.float32),
        grid=(n_m,),
        in_specs=[
            pl.BlockSpec((bm, d_model), lambda i: (i, 0)),
            pl.BlockSpec((d_model, hidden), lambda i: (0, 0)),
            pl.BlockSpec((1, hidden), lambda i: (0, 0)),
            pl.BlockSpec(memory_space=pl.ANY),
            pl.BlockSpec((1, d_model), lambda i: (0, 0)),
        ],
        out_specs=pl.BlockSpec((bm, d_model), lambda i: (i, 0)),
        scratch_shapes=[
            pltpu.VMEM((hidden, d_model), jnp.float32),
            pltpu.SemaphoreType.DMA((2,)),
        ],
        compiler_params=pltpu.CompilerParams(
            dimension_semantics=("arbitrary",),
            vmem_limit_bytes=int(0.95 * 64 * 1024 * 1024),
        ),
    )(x2d, w1, b1, w2, b2)

    return out2d.reshape(batch, seq, d_model)


def kernel(x, w1, b1, w2, b2):
    return _ffn(x, w1, b1, w2, b2)


# speedup vs baseline: 6.2000x; 1.0429x over previous
import functools

import jax
import jax.numpy as jnp
from jax.experimental import pallas as pl
from jax.experimental.pallas import tpu as pltpu


def _cdiv(a, b):
    return -(-a // b)


def _ffn_kernel(x_ref, w1_hbm, b1_ref, w2_hbm, b2_ref, o_ref, w1_v, w2_v, sem):
    hidden = w1_v.shape[1]
    hh = hidden // 2
    first = pl.program_id(0) == 0

    @pl.when(first)
    def _():
        pltpu.make_async_copy(w1_hbm.at[:, 0:hh], w1_v.at[:, 0:hh], sem.at[0]).start()
        pltpu.make_async_copy(w1_hbm.at[:, hh:], w1_v.at[:, hh:], sem.at[1]).start()
        pltpu.make_async_copy(w2_hbm.at[0:hh], w2_v.at[0:hh], sem.at[2]).start()
        pltpu.make_async_copy(w2_hbm.at[hh:], w2_v.at[hh:], sem.at[3]).start()

        x = x_ref[...]
        pltpu.make_async_copy(w1_v.at[:, 0:hh], w1_v.at[:, 0:hh], sem.at[0]).wait()
        h0 = jnp.dot(x, w1_v[:, 0:hh], preferred_element_type=jnp.float32)
        h0 = jnp.maximum(h0 + b1_ref[:, 0:hh], 0.0)
        pltpu.make_async_copy(w1_v.at[:, hh:], w1_v.at[:, hh:], sem.at[1]).wait()
        h1 = jnp.dot(x, w1_v[:, hh:], preferred_element_type=jnp.float32)
        h1 = jnp.maximum(h1 + b1_ref[:, hh:], 0.0)
        pltpu.make_async_copy(w2_v.at[0:hh], w2_v.at[0:hh], sem.at[2]).wait()
        y = jnp.dot(h0, w2_v[0:hh, :], preferred_element_type=jnp.float32)
        pltpu.make_async_copy(w2_v.at[hh:], w2_v.at[hh:], sem.at[3]).wait()
        y = y + jnp.dot(h1, w2_v[hh:, :], preferred_element_type=jnp.float32)
        o_ref[...] = y + b2_ref[...]

    @pl.when(jnp.logical_not(first))
    def _():
        h = jnp.dot(x_ref[...], w1_v[...], preferred_element_type=jnp.float32)
        h = jnp.maximum(h + b1_ref[...], 0.0)
        y = jnp.dot(h, w2_v[...], preferred_element_type=jnp.float32)
        o_ref[...] = y + b2_ref[...]


@functools.partial(jax.jit, static_argnames=("block_m",))
def _ffn(x, w1, b1, w2, b2, *, block_m=512):
    batch, seq, d_model = x.shape
    hidden = w1.shape[1]
    M = batch * seq

    x2d = x.reshape(M, d_model)
    bm = min(block_m, M)
    n_m = _cdiv(M, bm)

    out2d = pl.pallas_call(
        _ffn_kernel,
        out_shape=jax.ShapeDtypeStruct((M, d_model), jnp.float32),
        grid=(n_m,),
        in_specs=[
            pl.BlockSpec((bm, d_model), lambda i: (i, 0)),
            pl.BlockSpec(memory_space=pl.ANY),
            pl.BlockSpec((1, hidden), lambda i: (0, 0)),
            pl.BlockSpec(memory_space=pl.ANY),
            pl.BlockSpec((1, d_model), lambda i: (0, 0)),
        ],
        out_specs=pl.BlockSpec((bm, d_model), lambda i: (i, 0)),
        scratch_shapes=[
            pltpu.VMEM((d_model, hidden), jnp.float32),
            pltpu.VMEM((hidden, d_model), jnp.float32),
            pltpu.SemaphoreType.DMA((4,)),
        ],
        compiler_params=pltpu.CompilerParams(
            dimension_semantics=("arbitrary",),
            vmem_limit_bytes=int(0.95 * 64 * 1024 * 1024),
        ),
    )(x2d, w1, b1, w2, b2)

    return out2d.reshape(batch, seq, d_model)


def kernel(x, w1, b1, w2, b2):
    return _ffn(x, w1, b1, w2, b2)


# speedup vs baseline: 6.4034x; 1.0328x over previous
import functools

import jax
import jax.numpy as jnp
from jax.experimental import pallas as pl
from jax.experimental.pallas import tpu as pltpu


def _cdiv(a, b):
    return -(-a // b)


def _ffn_kernel(x_ref, w1_hbm, b1_ref, w2_hbm, b2_ref, o_ref, w1_v, w2_v, sem):
    hidden = w1_v.shape[1]
    nq = 4
    q = hidden // nq
    first = pl.program_id(0) == 0

    @pl.when(first)
    def _():
        for k in range(nq):
            pltpu.make_async_copy(w1_hbm.at[:, k * q:(k + 1) * q],
                                  w1_v.at[:, k * q:(k + 1) * q], sem.at[k]).start()
        for k in range(nq):
            pltpu.make_async_copy(w2_hbm.at[k * q:(k + 1) * q],
                                  w2_v.at[k * q:(k + 1) * q], sem.at[nq + k]).start()

        x = x_ref[...]
        hs = []
        for k in range(nq):
            sl = slice(k * q, (k + 1) * q)
            pltpu.make_async_copy(w1_v.at[:, sl], w1_v.at[:, sl], sem.at[k]).wait()
            hk = jnp.dot(x, w1_v[:, sl], preferred_element_type=jnp.float32)
            hs.append(jnp.maximum(hk + b1_ref[:, sl], 0.0))
        y = b2_ref[...]
        for k in range(nq):
            sl = slice(k * q, (k + 1) * q)
            pltpu.make_async_copy(w2_v.at[sl], w2_v.at[sl], sem.at[nq + k]).wait()
            y = y + jnp.dot(hs[k], w2_v[sl, :], preferred_element_type=jnp.float32)
        o_ref[...] = y

    @pl.when(jnp.logical_not(first))
    def _():
        h = jnp.dot(x_ref[...], w1_v[...], preferred_element_type=jnp.float32)
        h = jnp.maximum(h + b1_ref[...], 0.0)
        y = jnp.dot(h, w2_v[...], preferred_element_type=jnp.float32)
        o_ref[...] = y + b2_ref[...]


@functools.partial(jax.jit, static_argnames=("block_m",))
def _ffn(x, w1, b1, w2, b2, *, block_m=512):
    batch, seq, d_model = x.shape
    hidden = w1.shape[1]
    M = batch * seq

    x2d = x.reshape(M, d_model)
    bm = min(block_m, M)
    n_m = _cdiv(M, bm)

    out2d = pl.pallas_call(
        _ffn_kernel,
        out_shape=jax.ShapeDtypeStruct((M, d_model), jnp.float32),
        grid=(n_m,),
        in_specs=[
            pl.BlockSpec((bm, d_model), lambda i: (i, 0)),
            pl.BlockSpec(memory_space=pl.ANY),
            pl.BlockSpec((1, hidden), lambda i: (0, 0)),
            pl.BlockSpec(memory_space=pl.ANY),
            pl.BlockSpec((1, d_model), lambda i: (0, 0)),
        ],
        out_specs=pl.BlockSpec((bm, d_model), lambda i: (i, 0)),
        scratch_shapes=[
            pltpu.VMEM((d_model, hidden), jnp.float32),
            pltpu.VMEM((hidden, d_model), jnp.float32),
            pltpu.SemaphoreType.DMA((8,)),
        ],
        compiler_params=pltpu.CompilerParams(
            dimension_semantics=("arbitrary",),
            vmem_limit_bytes=int(0.95 * 64 * 1024 * 1024),
        ),
    )(x2d, w1, b1, w2, b2)

    return out2d.reshape(batch, seq, d_model)


def kernel(x, w1, b1, w2, b2):
    return _ffn(x, w1, b1, w2, b2)
